# baseline (speedup 1.0000x reference)
import functools

import jax
import jax.numpy as jnp
from jax import lax
from jax.experimental import pallas as pl
from jax.experimental.pallas import tpu as pltpu
from jax.experimental.pallas import tpu_sc as plsc

N_NODES = 10000
DIM = 128
NC = 2
NS = 16
NW = NC * NS
CHUNK = 128
N_PAD = 10240
ROWS_PER_TILE = N_PAD // NS
CNT_W = 16
BLK = 128

def _mesh():
    return plsc.VectorSubcoreMesh(
        core_axis_name="c", subcore_axis_name="s", num_cores=NC, num_subcores=NS)


def _deg_body(nchunk, pairs_hbm, zeros_hbm, out_hbm, idx_v, cnt_local):
    c = lax.axis_index("c")
    s = lax.axis_index("s")
    wid = c * NS + s
    pltpu.sync_copy(zeros_hbm, cnt_local)
    pltpu.sync_copy(pairs_hbm.at[pl.ds(wid * nchunk, nchunk)], idx_v)
    ones = jnp.ones((16,), jnp.float32)

    def body(j, carry):
        for g in range(CHUNK // 16):
            iv = idx_v[j, 1, pl.ds(g * 16, 16)]
            plsc.addupdate_scatter(cnt_local, [iv], ones)
        return carry

    lax.fori_loop(0, nchunk, body, 0)
    pltpu.sync_copy(cnt_local, out_hbm.at[wid])


NBUF = 2


def _gather_body(nchunk, pairs_hbm, g_hbm, zeros_hbm, out_hbm,
                 ibuf, rows, acc_sh, *sems):
    isems, gsems = sems[:NBUF], sems[NBUF:]
    c = lax.axis_index("c")
    s = lax.axis_index("s")
    wid = c * NS + s
    base = wid * nchunk
    pltpu.sync_copy(zeros_hbm, acc_sh.at[pl.ds(s * ROWS_PER_TILE, ROWS_PER_TILE)])
    plsc.subcore_barrier()

    def idx_wait(b):
        pltpu.make_async_copy(pairs_hbm.at[0], ibuf.at[b], isems[b]).wait()

    def gat_wait(b):
        pltpu.make_async_copy(g_hbm.at[pl.ds(0, CHUNK)], rows.at[b],
                              gsems[b]).wait()

    for b in range(NBUF):
        pltpu.async_copy(pairs_hbm.at[base + b], ibuf.at[b], isems[b])
    idx_wait(0)
    pltpu.async_copy(g_hbm.at[ibuf.at[0, 0]], rows.at[0], gsems[0])

    def body(jj, carry):
        for b in range(NBUF):
            j = jj * NBUF + b
            b1 = (b + 1) % NBUF
            gat_wait(b)

            @pl.when(j + 1 < nchunk)
            def _():
                idx_wait(b1)
                pltpu.async_copy(g_hbm.at[ibuf.at[b1, 0]], rows.at[b1],
                                 gsems[b1])

            pltpu.sync_copy(rows.at[b], acc_sh.at[ibuf.at[b, 1]], add=True)

            @pl.when(j + NBUF < nchunk)
            def _():
                pltpu.async_copy(pairs_hbm.at[base + j + NBUF], ibuf.at[b],
                                 isems[b])
        return carry

    lax.fori_loop(0, nchunk // NBUF, body, 0)
    plsc.subcore_barrier()
    pltpu.sync_copy(acc_sh.at[pl.ds(s * ROWS_PER_TILE, ROWS_PER_TILE)],
                    out_hbm.at[pl.ds(c * N_PAD + s * ROWS_PER_TILE, ROWS_PER_TILE)])


def _deg_call(pairs, zeros_deg, nchunk):
    k = pl.kernel(
        functools.partial(_deg_body, nchunk),
        out_type=jax.ShapeDtypeStruct((NW, N_PAD), jnp.float32),
        mesh=_mesh(),
        scratch_types=[
            pltpu.VMEM((nchunk, 2, CHUNK), jnp.int32),
            pltpu.VMEM((N_PAD,), jnp.float32),
        ],
        compiler_params=pltpu.CompilerParams(needs_layout_passes=False),
    )
    return k(pairs, zeros_deg)


def _gather_call(pairs, g, zeros_rows, nchunk):
    k = pl.kernel(
        functools.partial(_gather_body, nchunk),
        out_type=jax.ShapeDtypeStruct((NC * N_PAD, DIM), jnp.float32),
        mesh=_mesh(),
        scratch_types=[
            pltpu.VMEM((NBUF, 2, CHUNK), jnp.int32),
            pltpu.VMEM((NBUF, CHUNK, DIM), jnp.float32),
            pltpu.VMEM_SHARED((N_PAD, DIM), jnp.float32),
        ] + [pltpu.SemaphoreType.DMA] * (2 * NBUF),
    )
    return k(pairs, g, zeros_rows)


def _dinv(cnt):
    deg = jnp.sum(cnt, axis=0) + 1.0
    return lax.rsqrt(deg)[:, None]


def _tc1_body(x_ref, w1_ref, cnt_ref, g_ref):
    dinv = _dinv(cnt_ref[...])
    g_ref[...] = jnp.dot(x_ref[...], w1_ref[...],
                         preferred_element_type=jnp.float32) * dinv


def _tc2_body(acc_ref, g1_ref, cnt_ref, w2_ref, b1_ref, g2_ref):
    dinv = _dinv(cnt_ref[...])
    a = acc_ref[0] + acc_ref[1] + g1_ref[...]
    h = jnp.maximum(a * dinv + b1_ref[...], 0.0)
    g2_ref[...] = jnp.dot(h, w2_ref[...],
                          preferred_element_type=jnp.float32) * dinv


def _tc3_body(acc_ref, g2_ref, cnt_ref, b2_ref, nb_ref, pv_ref, wn_ref, wt_ref,
              out_ref):
    dinv = _dinv(cnt_ref[...])
    a = acc_ref[0] + acc_ref[1] + g2_ref[...]
    out_ref[...] = (a * dinv + b2_ref[...]
                    + jnp.dot(nb_ref[...], wn_ref[...],
                              preferred_element_type=jnp.float32)
                    + jnp.dot(pv_ref[...], wt_ref[...],
                              preferred_element_type=jnp.float32))


_ROW = pl.BlockSpec((BLK, DIM), lambda i: (i, 0))
_MAT = pl.BlockSpec((DIM, DIM), lambda i: (0, 0))
_CNT = pl.BlockSpec((NW, BLK), lambda i: (0, i))
_ACC = pl.BlockSpec((2, BLK, DIM), lambda i: (0, i, 0))
_BIAS = pl.BlockSpec((1, DIM), lambda i: (0, 0))
_GRID = (N_PAD // BLK,)
_OUT_ROWS = jax.ShapeDtypeStruct((N_PAD, DIM), jnp.float32)


def _tc1(x_pad, W1, counts):
    return pl.pallas_call(
        _tc1_body, grid=_GRID,
        in_specs=[_ROW, _MAT, _CNT], out_specs=_ROW,
        out_shape=_OUT_ROWS)(x_pad, W1, counts)


def _tc2(acc1, g1, counts, W2, b1):
    return pl.pallas_call(
        _tc2_body, grid=_GRID,
        in_specs=[_ACC, _ROW, _CNT, _MAT, _BIAS], out_specs=_ROW,
        out_shape=_OUT_ROWS)(acc1, g1, counts, W2, b1)


def _tc3(acc2, g2, counts, b2, nb, pv, Wn, Wt):
    return pl.pallas_call(
        _tc3_body, grid=_GRID,
        in_specs=[_ACC, _ROW, _CNT, _BIAS, _ROW, _ROW, _MAT, _MAT],
        out_specs=_ROW,
        out_shape=_OUT_ROWS)(acc2, g2, counts, b2, nb, pv, Wn, Wt)


def kernel(x, edge_index, neighbor_features, prev_time_features,
           W1, b1, W2, b2, Wn, Wt):
    E = edge_index.shape[1]
    n = -(-E // (NW * CHUNK))
    nchunk = -(-n // NBUF) * NBUF
    e_pad = nchunk * NW * CHUNK
    ei = edge_index.astype(jnp.int32)
    pad = jnp.full((e_pad - E,), N_NODES, jnp.int32)
    src2 = jnp.concatenate([ei[0], pad]).reshape(nchunk * NW, CHUNK)
    dst2 = jnp.concatenate([ei[1], pad]).reshape(nchunk * NW, CHUNK)
    pairs = jnp.stack([src2, dst2], axis=1)

    rpad = ((0, N_PAD - N_NODES), (0, 0))
    x_pad = jnp.pad(x, rpad)
    nb_pad = jnp.pad(neighbor_features, rpad)
    pv_pad = jnp.pad(prev_time_features, rpad)
    zeros_deg = jnp.zeros((N_PAD,), jnp.float32)
    zeros_rows = jnp.zeros((ROWS_PER_TILE, DIM), jnp.float32)

    counts = _deg_call(pairs, zeros_deg, nchunk)
    g1 = _tc1(x_pad, W1, counts)
    acc1 = _gather_call(pairs, g1, zeros_rows, nchunk).reshape(NC, N_PAD, DIM)
    g2 = _tc2(acc1, g1, counts, W2, b1.reshape(1, DIM))
    acc2 = _gather_call(pairs, g2, zeros_rows, nchunk).reshape(NC, N_PAD, DIM)
    out = _tc3(acc2, g2, counts, b2.reshape(1, DIM), nb_pad, pv_pad, Wn, Wt)
    return out[:N_NODES]


# speedup vs baseline: 1.0772x; 1.0772x over previous
import functools

import jax
import jax.numpy as jnp
from jax import lax
from jax.experimental import pallas as pl
from jax.experimental.pallas import tpu as pltpu
from jax.experimental.pallas import tpu_sc as plsc

N_NODES = 10000
DIM = 128
NC = 2
NS = 16
NW = NC * NS
CHUNK = 128
N_PAD = 10240
ROWS_PER_TILE = N_PAD // NS
CNT_W = 16
BLK = 128

def _mesh():
    return plsc.VectorSubcoreMesh(
        core_axis_name="c", subcore_axis_name="s", num_cores=NC, num_subcores=NS)


def _deg_body(nchunk, pairs_hbm, zeros_hbm, out_hbm, idx_v, cnt_local):
    c = lax.axis_index("c")
    s = lax.axis_index("s")
    wid = c * NS + s
    pltpu.sync_copy(zeros_hbm, cnt_local)
    pltpu.sync_copy(pairs_hbm.at[pl.ds(wid * nchunk, nchunk)], idx_v)
    ones = jnp.ones((16,), jnp.float32)

    def body(j, carry):
        for g in range(CHUNK // 16):
            iv = idx_v[j, 1, pl.ds(g * 16, 16)]
            plsc.addupdate_scatter(cnt_local, [iv], ones)
        return carry

    lax.fori_loop(0, nchunk, body, 0)
    pltpu.sync_copy(cnt_local, out_hbm.at[wid])


NBUF = 2


def _gather_body(n0, n1, pairs_hbm, g_hbm, zeros_hbm, out_hbm,
                 ibuf, rows, acc_sh, *sems):
    isems, gsems = sems[:NBUF], sems[NBUF:]
    c = lax.axis_index("c")
    s = lax.axis_index("s")
    my_n = jnp.where(c == 0, n0, n1)
    base = jnp.where(c == 0, s * n0, NS * n0 + s * n1)
    pltpu.sync_copy(zeros_hbm, acc_sh.at[pl.ds(s * ROWS_PER_TILE, ROWS_PER_TILE)])
    plsc.subcore_barrier()

    def idx_wait(b):
        pltpu.make_async_copy(pairs_hbm.at[0], ibuf.at[b], isems[b]).wait()

    def gat_wait(b):
        pltpu.make_async_copy(g_hbm.at[pl.ds(0, CHUNK)], rows.at[b],
                              gsems[b]).wait()

    for b in range(NBUF):
        pltpu.async_copy(pairs_hbm.at[base + b], ibuf.at[b], isems[b])
    idx_wait(0)
    pltpu.async_copy(g_hbm.at[ibuf.at[0, 0]], rows.at[0], gsems[0])

    def body(jj, carry):
        for b in range(NBUF):
            j = jj * NBUF + b
            b1 = (b + 1) % NBUF
            gat_wait(b)

            @pl.when(j + 1 < my_n)
            def _():
                idx_wait(b1)
                pltpu.async_copy(g_hbm.at[ibuf.at[b1, 0]], rows.at[b1],
                                 gsems[b1])

            pltpu.sync_copy(rows.at[b], acc_sh.at[ibuf.at[b, 1]], add=True)

            @pl.when(j + NBUF < my_n)
            def _():
                pltpu.async_copy(pairs_hbm.at[base + j + NBUF], ibuf.at[b],
                                 isems[b])
        return carry

    lax.fori_loop(0, my_n // NBUF, body, 0)
    plsc.subcore_barrier()
    pltpu.sync_copy(acc_sh.at[pl.ds(s * ROWS_PER_TILE, ROWS_PER_TILE)],
                    out_hbm.at[pl.ds(c * N_PAD + s * ROWS_PER_TILE, ROWS_PER_TILE)])


def _deg_call(pairs, zeros_deg, nchunk):
    k = pl.kernel(
        functools.partial(_deg_body, nchunk),
        out_type=jax.ShapeDtypeStruct((NW, N_PAD), jnp.float32),
        mesh=_mesh(),
        scratch_types=[
            pltpu.VMEM((nchunk, 2, CHUNK), jnp.int32),
            pltpu.VMEM((N_PAD,), jnp.float32),
        ],
        compiler_params=pltpu.CompilerParams(needs_layout_passes=False),
    )
    return k(pairs, zeros_deg)


def _gather_call(pairs, g, zeros_rows, n0, n1):
    k = pl.kernel(
        functools.partial(_gather_body, n0, n1),
        out_type=jax.ShapeDtypeStruct((NC * N_PAD, DIM), jnp.float32),
        mesh=_mesh(),
        scratch_types=[
            pltpu.VMEM((NBUF, 2, CHUNK), jnp.int32),
            pltpu.VMEM((NBUF, CHUNK, DIM), jnp.float32),
            pltpu.VMEM_SHARED((N_PAD, DIM), jnp.float32),
        ] + [pltpu.SemaphoreType.DMA] * (2 * NBUF),
    )
    return k(pairs, g, zeros_rows)


def _dinv(cnt):
    deg = jnp.sum(cnt, axis=0) + 1.0
    return lax.rsqrt(deg)[:, None]


def _tc1_body(x_ref, w1_ref, cnt_ref, g_ref):
    dinv = _dinv(cnt_ref[...])
    g_ref[...] = jnp.dot(x_ref[...], w1_ref[...],
                         preferred_element_type=jnp.float32) * dinv


def _tc2_body(acc_ref, g1_ref, cnt_ref, w2_ref, b1_ref, g2_ref):
    dinv = _dinv(cnt_ref[...])
    a = acc_ref[0] + acc_ref[1] + g1_ref[...]
    h = jnp.maximum(a * dinv + b1_ref[...], 0.0)
    g2_ref[...] = jnp.dot(h, w2_ref[...],
                          preferred_element_type=jnp.float32) * dinv


def _tc3_body(acc_ref, g2_ref, cnt_ref, b2_ref, nb_ref, pv_ref, wn_ref, wt_ref,
              out_ref):
    dinv = _dinv(cnt_ref[...])
    a = acc_ref[0] + acc_ref[1] + g2_ref[...]
    out_ref[...] = (a * dinv + b2_ref[...]
                    + jnp.dot(nb_ref[...], wn_ref[...],
                              preferred_element_type=jnp.float32)
                    + jnp.dot(pv_ref[...], wt_ref[...],
                              preferred_element_type=jnp.float32))


_ROW = pl.BlockSpec((BLK, DIM), lambda i: (i, 0))
_MAT = pl.BlockSpec((DIM, DIM), lambda i: (0, 0))
_CNT = pl.BlockSpec((NW, BLK), lambda i: (0, i))
_ACC = pl.BlockSpec((2, BLK, DIM), lambda i: (0, i, 0))
_BIAS = pl.BlockSpec((1, DIM), lambda i: (0, 0))
_GRID = (N_PAD // BLK,)
_OUT_ROWS = jax.ShapeDtypeStruct((N_PAD, DIM), jnp.float32)


def _tc1(x_pad, W1, counts):
    return pl.pallas_call(
        _tc1_body, grid=_GRID,
        in_specs=[_ROW, _MAT, _CNT], out_specs=_ROW,
        out_shape=_OUT_ROWS)(x_pad, W1, counts)


def _tc2(acc1, g1, counts, W2, b1):
    return pl.pallas_call(
        _tc2_body, grid=_GRID,
        in_specs=[_ACC, _ROW, _CNT, _MAT, _BIAS], out_specs=_ROW,
        out_shape=_OUT_ROWS)(acc1, g1, counts, W2, b1)


def _tc3(acc2, g2, counts, b2, nb, pv, Wn, Wt):
    return pl.pallas_call(
        _tc3_body, grid=_GRID,
        in_specs=[_ACC, _ROW, _CNT, _BIAS, _ROW, _ROW, _MAT, _MAT],
        out_specs=_ROW,
        out_shape=_OUT_ROWS)(acc2, g2, counts, b2, nb, pv, Wn, Wt)


def kernel(x, edge_index, neighbor_features, prev_time_features,
           W1, b1, W2, b2, Wn, Wt):
    E = edge_index.shape[1]
    tch = -(-(-(-E // CHUNK)) // (4 * NW)) * 4 * NW
    nchunk = tch // NW
    n1 = tch // (4 * NS)
    n0 = 3 * n1
    e_pad = tch * CHUNK
    ei = edge_index.astype(jnp.int32)
    pad = jnp.full((e_pad - E,), N_NODES, jnp.int32)
    src2 = jnp.concatenate([ei[0], pad]).reshape(tch, CHUNK)
    dst2 = jnp.concatenate([ei[1], pad]).reshape(tch, CHUNK)
    pairs = jnp.stack([src2, dst2], axis=1)

    rpad = ((0, N_PAD - N_NODES), (0, 0))
    x_pad = jnp.pad(x, rpad)
    nb_pad = jnp.pad(neighbor_features, rpad)
    pv_pad = jnp.pad(prev_time_features, rpad)
    zeros_deg = jnp.zeros((N_PAD,), jnp.float32)
    zeros_rows = jnp.zeros((ROWS_PER_TILE, DIM), jnp.float32)

    counts = _deg_call(pairs, zeros_deg, nchunk)
    g1 = _tc1(x_pad, W1, counts)
    acc1 = _gather_call(pairs, g1, zeros_rows, n0, n1).reshape(NC, N_PAD, DIM)
    g2 = _tc2(acc1, g1, counts, W2, b1.reshape(1, DIM))
    acc2 = _gather_call(pairs, g2, zeros_rows, n0, n1).reshape(NC, N_PAD, DIM)
    out = _tc3(acc2, g2, counts, b2.reshape(1, DIM), nb_pad, pv_pad, Wn, Wt)
    return out[:N_NODES]


# speedup vs baseline: 1.1336x; 1.0524x over previous
import functools

import jax
import jax.numpy as jnp
from jax import lax
from jax.experimental import pallas as pl
from jax.experimental.pallas import tpu as pltpu
from jax.experimental.pallas import tpu_sc as plsc

N_NODES = 10000
DIM = 128
NC = 2
NS = 16
NW = NC * NS
CHUNK = 128
N_PAD = 10240
ROWS_PER_TILE = N_PAD // NS
CNT_W = 16
BLK = 128

def _mesh():
    return plsc.VectorSubcoreMesh(
        core_axis_name="c", subcore_axis_name="s", num_cores=NC, num_subcores=NS)


def _deg_body(nchunk, pairs_hbm, zeros_hbm, out_hbm, idx_v, cnt_local):
    c = lax.axis_index("c")
    s = lax.axis_index("s")
    wid = c * NS + s
    pltpu.sync_copy(zeros_hbm, cnt_local)
    pltpu.sync_copy(pairs_hbm.at[pl.ds(wid * nchunk, nchunk)], idx_v)
    ones = jnp.ones((16,), jnp.float32)

    def body(j, carry):
        for g in range(CHUNK // 16):
            iv = idx_v[j, 1, pl.ds(g * 16, 16)]
            plsc.addupdate_scatter(cnt_local, [iv], ones)
        return carry

    lax.fori_loop(0, nchunk, body, 0)
    pltpu.sync_copy(cnt_local, out_hbm.at[wid])


NBUF = 2


def _gather_body(n0, n1, pairs_hbm, g_hbm, zeros_hbm, out_hbm,
                 ibuf, rows, acc_sh, *sems):
    isems, gsems = sems[:NBUF], sems[NBUF:]
    c = lax.axis_index("c")
    s = lax.axis_index("s")
    my_n = jnp.where(c == 0, n0, n1)
    base = jnp.where(c == 0, s * n0, NS * n0 + s * n1)
    pltpu.sync_copy(zeros_hbm, acc_sh.at[pl.ds(s * ROWS_PER_TILE, ROWS_PER_TILE)])
    plsc.subcore_barrier()

    def idx_wait(b):
        pltpu.make_async_copy(pairs_hbm.at[0], ibuf.at[b], isems[b]).wait()

    def gat_wait(b):
        pltpu.make_async_copy(g_hbm.at[pl.ds(0, CHUNK)], rows.at[b],
                              gsems[b]).wait()

    for b in range(NBUF):
        pltpu.async_copy(pairs_hbm.at[base + b], ibuf.at[b], isems[b])
    idx_wait(0)
    pltpu.async_copy(g_hbm.at[ibuf.at[0, 0]], rows.at[0], gsems[0])

    def body(jj, carry):
        for b in range(NBUF):
            j = jj * NBUF + b
            b1 = (b + 1) % NBUF
            gat_wait(b)

            @pl.when(j + 1 < my_n)
            def _():
                idx_wait(b1)
                pltpu.async_copy(g_hbm.at[ibuf.at[b1, 0]], rows.at[b1],
                                 gsems[b1])

            pltpu.sync_copy(rows.at[b], acc_sh.at[ibuf.at[b, 1]], add=True)

            @pl.when(j + NBUF < my_n)
            def _():
                pltpu.async_copy(pairs_hbm.at[base + j + NBUF], ibuf.at[b],
                                 isems[b])
        return carry

    lax.fori_loop(0, my_n // NBUF, body, 0)
    plsc.subcore_barrier()
    pltpu.sync_copy(acc_sh.at[pl.ds(s * ROWS_PER_TILE, ROWS_PER_TILE)],
                    out_hbm.at[pl.ds(c * N_PAD + s * ROWS_PER_TILE, ROWS_PER_TILE)])


def _deg_call(pairs, zeros_deg, nchunk):
    k = pl.kernel(
        functools.partial(_deg_body, nchunk),
        out_type=jax.ShapeDtypeStruct((NW, N_PAD), jnp.float32),
        mesh=_mesh(),
        scratch_types=[
            pltpu.VMEM((nchunk, 2, CHUNK), jnp.int32),
            pltpu.VMEM((N_PAD,), jnp.float32),
        ],
        compiler_params=pltpu.CompilerParams(needs_layout_passes=False),
    )
    return k(pairs, zeros_deg)


def _gather_call(pairs, g, zeros_rows, n0, n1):
    k = pl.kernel(
        functools.partial(_gather_body, n0, n1),
        out_type=jax.ShapeDtypeStruct((NC * N_PAD, DIM), jnp.float32),
        mesh=_mesh(),
        scratch_types=[
            pltpu.VMEM((NBUF, 2, CHUNK), jnp.int32),
            pltpu.VMEM((NBUF, CHUNK, DIM), jnp.float32),
            pltpu.VMEM_SHARED((N_PAD, DIM), jnp.float32),
        ] + [pltpu.SemaphoreType.DMA] * (2 * NBUF),
    )
    return k(pairs, g, zeros_rows)


def _dinv(cnt):
    deg = jnp.sum(cnt, axis=0) + 1.0
    return lax.rsqrt(deg)[:, None]


def _tc1_body(x_ref, w1_ref, cnt_ref, g_ref):
    dinv = _dinv(cnt_ref[...])
    g_ref[...] = jnp.dot(x_ref[...], w1_ref[...],
                         preferred_element_type=jnp.float32) * dinv


def _tc2_body(acc_ref, g1_ref, cnt_ref, w2_ref, b1_ref, g2_ref):
    dinv = _dinv(cnt_ref[...])
    a = acc_ref[0] + acc_ref[1] + g1_ref[...]
    h = jnp.maximum(a * dinv + b1_ref[...], 0.0)
    g2_ref[...] = jnp.dot(h, w2_ref[...],
                          preferred_element_type=jnp.float32) * dinv


def _tc3_body(acc_ref, g2_ref, cnt_ref, b2_ref, nb_ref, pv_ref, wn_ref, wt_ref,
              out_ref):
    dinv = _dinv(cnt_ref[...])
    a = acc_ref[0] + acc_ref[1] + g2_ref[...]
    out_ref[...] = (a * dinv + b2_ref[...]
                    + jnp.dot(nb_ref[...], wn_ref[...],
                              preferred_element_type=jnp.float32)
                    + jnp.dot(pv_ref[...], wt_ref[...],
                              preferred_element_type=jnp.float32))


_ROW = pl.BlockSpec((BLK, DIM), lambda i: (i, 0))
_MAT = pl.BlockSpec((DIM, DIM), lambda i: (0, 0))
_CNT = pl.BlockSpec((NW, BLK), lambda i: (0, i))
_ACC = pl.BlockSpec((2, BLK, DIM), lambda i: (0, i, 0))
_BIAS = pl.BlockSpec((1, DIM), lambda i: (0, 0))
_GRID = (N_PAD // BLK,)
_OUT_ROWS = jax.ShapeDtypeStruct((N_PAD, DIM), jnp.float32)


def _tc1(x_pad, W1, counts):
    return pl.pallas_call(
        _tc1_body, grid=_GRID,
        in_specs=[_ROW, _MAT, _CNT], out_specs=_ROW,
        out_shape=_OUT_ROWS)(x_pad, W1, counts)


def _tc2(acc1, g1, counts, W2, b1):
    return pl.pallas_call(
        _tc2_body, grid=_GRID,
        in_specs=[_ACC, _ROW, _CNT, _MAT, _BIAS], out_specs=_ROW,
        out_shape=_OUT_ROWS)(acc1, g1, counts, W2, b1)


def _tc3(acc2, g2, counts, b2, nb, pv, Wn, Wt):
    return pl.pallas_call(
        _tc3_body, grid=_GRID,
        in_specs=[_ACC, _ROW, _CNT, _BIAS, _ROW, _ROW, _MAT, _MAT],
        out_specs=_ROW,
        out_shape=_OUT_ROWS)(acc2, g2, counts, b2, nb, pv, Wn, Wt)


def kernel(x, edge_index, neighbor_features, prev_time_features,
           W1, b1, W2, b2, Wn, Wt):
    E = edge_index.shape[1]
    tch = -(-(-(-E // CHUNK)) // (4 * NW)) * 4 * NW
    nchunk = tch // NW
    n1 = 2 * (-(-(3 * tch) // (20 * NS)) // 2)
    n0 = tch // NS - n1
    e_pad = tch * CHUNK
    ei = edge_index.astype(jnp.int32)
    pad = jnp.full((e_pad - E,), N_NODES, jnp.int32)
    src2 = jnp.concatenate([ei[0], pad]).reshape(tch, CHUNK)
    dst2 = jnp.concatenate([ei[1], pad]).reshape(tch, CHUNK)
    pairs = jnp.stack([src2, dst2], axis=1)

    rpad = ((0, N_PAD - N_NODES), (0, 0))
    x_pad = jnp.pad(x, rpad)
    nb_pad = jnp.pad(neighbor_features, rpad)
    pv_pad = jnp.pad(prev_time_features, rpad)
    zeros_deg = jnp.zeros((N_PAD,), jnp.float32)
    zeros_rows = jnp.zeros((ROWS_PER_TILE, DIM), jnp.float32)

    counts = _deg_call(pairs, zeros_deg, nchunk)
    g1 = _tc1(x_pad, W1, counts)
    acc1 = _gather_call(pairs, g1, zeros_rows, n0, n1).reshape(NC, N_PAD, DIM)
    g2 = _tc2(acc1, g1, counts, W2, b1.reshape(1, DIM))
    acc2 = _gather_call(pairs, g2, zeros_rows, n0, n1).reshape(NC, N_PAD, DIM)
    out = _tc3(acc2, g2, counts, b2.reshape(1, DIM), nb_pad, pv_pad, Wn, Wt)
    return out[:N_NODES]


# speedup vs baseline: 1.2130x; 1.0700x over previous
import functools

import jax
import jax.numpy as jnp
from jax import lax
from jax.experimental import pallas as pl
from jax.experimental.pallas import tpu as pltpu
from jax.experimental.pallas import tpu_sc as plsc

N_NODES = 10000
DIM = 128
NC = 2
NS = 16
NW = NC * NS
CHUNK = 128
N_PAD = 10240
ROWS_PER_TILE = N_PAD // NS
CNT_W = 16
BLK = 128

def _mesh():
    return plsc.VectorSubcoreMesh(
        core_axis_name="c", subcore_axis_name="s", num_cores=NC, num_subcores=NS)


def _deg_body(nchunk, pairs_hbm, out_hbm, idx_v, cnt_local):
    c = lax.axis_index("c")
    s = lax.axis_index("s")
    wid = c * NS + s
    zeros16 = jnp.zeros((16,), jnp.float32)

    def zbody(j, carry):
        cnt_local[pl.ds(j * 16, 16)] = zeros16
        return carry

    lax.fori_loop(0, N_PAD // 16, zbody, 0)
    pltpu.sync_copy(pairs_hbm.at[pl.ds(wid * nchunk, nchunk)], idx_v)
    ones = jnp.ones((16,), jnp.float32)

    def body(j, carry):
        for g in range(CHUNK // 16):
            iv = idx_v[j, 1, pl.ds(g * 16, 16)]
            plsc.addupdate_scatter(cnt_local, [iv], ones)
        return carry

    lax.fori_loop(0, nchunk, body, 0)
    pltpu.sync_copy(cnt_local, out_hbm.at[wid])


NBUF = 2


def _gather_body(n0, n1, pairs_hbm, g_hbm, out_hbm,
                 ibuf, rows, acc_sh, *sems):
    isems, gsems = sems[:NBUF], sems[NBUF:]
    c = lax.axis_index("c")
    s = lax.axis_index("s")
    my_n = jnp.where(c == 0, n0, n1)
    base = jnp.where(c == 0, s * n0, NS * n0 + s * n1)

    zeros16 = jnp.zeros((16,), jnp.float32)

    def zrow(r, carry):
        for k in range(DIM // 16):
            rows[0, r, pl.ds(k * 16, 16)] = zeros16
        return carry

    lax.fori_loop(0, CHUNK, zrow, 0)
    for k in range(ROWS_PER_TILE // CHUNK):
        pltpu.sync_copy(
            rows.at[0],
            acc_sh.at[pl.ds(s * ROWS_PER_TILE + k * CHUNK, CHUNK)])
    plsc.subcore_barrier()

    def idx_wait(b):
        pltpu.make_async_copy(pairs_hbm.at[0], ibuf.at[b], isems[b]).wait()

    def gat_wait(b):
        pltpu.make_async_copy(g_hbm.at[pl.ds(0, CHUNK)], rows.at[b],
                              gsems[b]).wait()

    for b in range(NBUF):
        pltpu.async_copy(pairs_hbm.at[base + b], ibuf.at[b], isems[b])
    idx_wait(0)
    pltpu.async_copy(g_hbm.at[ibuf.at[0, 0]], rows.at[0], gsems[0])

    def body(jj, carry):
        for b in range(NBUF):
            j = jj * NBUF + b
            b1 = (b + 1) % NBUF
            gat_wait(b)

            @pl.when(j + 1 < my_n)
            def _():
                idx_wait(b1)
                pltpu.async_copy(g_hbm.at[ibuf.at[b1, 0]], rows.at[b1],
                                 gsems[b1])

            pltpu.sync_copy(rows.at[b], acc_sh.at[ibuf.at[b, 1]], add=True)

            @pl.when(j + NBUF < my_n)
            def _():
                pltpu.async_copy(pairs_hbm.at[base + j + NBUF], ibuf.at[b],
                                 isems[b])
        return carry

    lax.fori_loop(0, my_n // NBUF, body, 0)
    plsc.subcore_barrier()
    pltpu.sync_copy(acc_sh.at[pl.ds(s * ROWS_PER_TILE, ROWS_PER_TILE)],
                    out_hbm.at[pl.ds(c * N_PAD + s * ROWS_PER_TILE, ROWS_PER_TILE)])


def _deg_call(pairs, nchunk):
    k = pl.kernel(
        functools.partial(_deg_body, nchunk),
        out_type=jax.ShapeDtypeStruct((NW, N_PAD), jnp.float32),
        mesh=_mesh(),
        scratch_types=[
            pltpu.VMEM((nchunk, 2, CHUNK), jnp.int32),
            pltpu.VMEM((N_PAD,), jnp.float32),
        ],
        compiler_params=pltpu.CompilerParams(needs_layout_passes=False),
    )
    return k(pairs)


def _gather_call(pairs, g, n0, n1):
    k = pl.kernel(
        functools.partial(_gather_body, n0, n1),
        out_type=jax.ShapeDtypeStruct((NC * N_PAD, DIM), jnp.float32),
        mesh=_mesh(),
        scratch_types=[
            pltpu.VMEM((NBUF, 2, CHUNK), jnp.int32),
            pltpu.VMEM((NBUF, CHUNK, DIM), jnp.float32),
            pltpu.VMEM_SHARED((N_PAD, DIM), jnp.float32),
        ] + [pltpu.SemaphoreType.DMA] * (2 * NBUF),
    )
    return k(pairs, g)


def _dinv(cnt):
    deg = jnp.sum(cnt, axis=0) + 1.0
    return lax.rsqrt(deg)[:, None]


def _tc1_body(x_ref, w1_ref, cnt_ref, g_ref):
    dinv = _dinv(cnt_ref[...])
    g_ref[...] = jnp.dot(x_ref[...], w1_ref[...],
                         preferred_element_type=jnp.float32) * dinv


def _tc2_body(acc_ref, g1_ref, cnt_ref, w2_ref, b1_ref, g2_ref):
    dinv = _dinv(cnt_ref[...])
    a = acc_ref[0] + acc_ref[1] + g1_ref[...]
    h = jnp.maximum(a * dinv + b1_ref[...], 0.0)
    g2_ref[...] = jnp.dot(h, w2_ref[...],
                          preferred_element_type=jnp.float32) * dinv


def _tc3_body(acc_ref, g2_ref, cnt_ref, b2_ref, nb_ref, pv_ref, wn_ref, wt_ref,
              out_ref):
    dinv = _dinv(cnt_ref[...])
    a = acc_ref[0] + acc_ref[1] + g2_ref[...]
    out_ref[...] = (a * dinv + b2_ref[...]
                    + jnp.dot(nb_ref[...], wn_ref[...],
                              preferred_element_type=jnp.float32)
                    + jnp.dot(pv_ref[...], wt_ref[...],
                              preferred_element_type=jnp.float32))


_ROW = pl.BlockSpec((BLK, DIM), lambda i: (i, 0))
_MAT = pl.BlockSpec((DIM, DIM), lambda i: (0, 0))
_CNT = pl.BlockSpec((NW, BLK), lambda i: (0, i))
_ACC = pl.BlockSpec((2, BLK, DIM), lambda i: (0, i, 0))
_BIAS = pl.BlockSpec((1, DIM), lambda i: (0, 0))
_GRID = (N_PAD // BLK,)
_OUT_ROWS = jax.ShapeDtypeStruct((N_PAD, DIM), jnp.float32)


def _tc1(x_pad, W1, counts):
    return pl.pallas_call(
        _tc1_body, grid=_GRID,
        in_specs=[_ROW, _MAT, _CNT], out_specs=_ROW,
        out_shape=_OUT_ROWS)(x_pad, W1, counts)


def _tc2(acc1, g1, counts, W2, b1):
    return pl.pallas_call(
        _tc2_body, grid=_GRID,
        in_specs=[_ACC, _ROW, _CNT, _MAT, _BIAS], out_specs=_ROW,
        out_shape=_OUT_ROWS)(acc1, g1, counts, W2, b1)


def _tc3(acc2, g2, counts, b2, nb, pv, Wn, Wt):
    return pl.pallas_call(
        _tc3_body, grid=_GRID,
        in_specs=[_ACC, _ROW, _CNT, _BIAS, _ROW, _ROW, _MAT, _MAT],
        out_specs=_ROW,
        out_shape=_OUT_ROWS)(acc2, g2, counts, b2, nb, pv, Wn, Wt)


def kernel(x, edge_index, neighbor_features, prev_time_features,
           W1, b1, W2, b2, Wn, Wt):
    E = edge_index.shape[1]
    tch = -(-(-(-E // CHUNK)) // (4 * NW)) * 4 * NW
    nchunk = tch // NW
    n1 = 2 * (-(-(3 * tch) // (20 * NS)) // 2)
    n0 = tch // NS - n1
    e_pad = tch * CHUNK
    ei = edge_index.astype(jnp.int32)
    pad = jnp.full((e_pad - E,), N_NODES, jnp.int32)
    src2 = jnp.concatenate([ei[0], pad]).reshape(tch, CHUNK)
    dst2 = jnp.concatenate([ei[1], pad]).reshape(tch, CHUNK)
    pairs = jnp.stack([src2, dst2], axis=1)

    rpad = ((0, N_PAD - N_NODES), (0, 0))
    x_pad = jnp.pad(x, rpad)
    nb_pad = jnp.pad(neighbor_features, rpad)
    pv_pad = jnp.pad(prev_time_features, rpad)
    counts = _deg_call(pairs, nchunk)
    g1 = _tc1(x_pad, W1, counts)
    acc1 = _gather_call(pairs, g1, n0, n1).reshape(NC, N_PAD, DIM)
    g2 = _tc2(acc1, g1, counts, W2, b1.reshape(1, DIM))
    acc2 = _gather_call(pairs, g2, n0, n1).reshape(NC, N_PAD, DIM)
    out = _tc3(acc2, g2, counts, b2.reshape(1, DIM), nb_pad, pv_pad, Wn, Wt)
    return out[:N_NODES]


# speedup vs baseline: 1.2136x; 1.0005x over previous
import functools

import jax
import jax.numpy as jnp
from jax import lax
from jax.experimental import pallas as pl
from jax.experimental.pallas import tpu as pltpu
from jax.experimental.pallas import tpu_sc as plsc

N_NODES = 10000
DIM = 128
NC = 2
NS = 16
NW = NC * NS
CHUNK = 128
N_PAD = 10240
ROWS_PER_TILE = N_PAD // NS
CNT_W = 16
BLK = 128

def _mesh():
    return plsc.VectorSubcoreMesh(
        core_axis_name="c", subcore_axis_name="s", num_cores=NC, num_subcores=NS)


def _deg_body(nchunk, pairs_hbm, out_hbm, idx_v, cnt_local):
    c = lax.axis_index("c")
    s = lax.axis_index("s")
    wid = c * NS + s
    zeros16 = jnp.zeros((16,), jnp.float32)

    def zbody(j, carry):
        cnt_local[pl.ds(j * 16, 16)] = zeros16
        return carry

    lax.fori_loop(0, N_PAD // 16, zbody, 0)
    pltpu.sync_copy(pairs_hbm.at[pl.ds(wid * nchunk, nchunk)], idx_v)
    ones = jnp.ones((16,), jnp.float32)

    def body(j, carry):
        for g in range(CHUNK // 16):
            iv = idx_v[j, 1, pl.ds(g * 16, 16)]
            plsc.addupdate_scatter(cnt_local, [iv], ones)
        return carry

    lax.fori_loop(0, nchunk, body, 0)
    pltpu.sync_copy(cnt_local, out_hbm.at[wid])


NBUF = 2


def _gather_body(n0, n1, pairs_hbm, g_hbm, out_hbm,
                 ibuf, rows, acc_sh, *sems):
    isems, gsems = sems[:NBUF], sems[NBUF:]
    c = lax.axis_index("c")
    s = lax.axis_index("s")
    my_n = jnp.where(c == 0, n0, n1)
    base = jnp.where(c == 0, s * n0, NS * n0 + s * n1)

    zeros16 = jnp.zeros((16,), jnp.float32)

    def zrow(r, carry):
        for k in range(DIM // 16):
            rows[0, r, pl.ds(k * 16, 16)] = zeros16
        return carry

    lax.fori_loop(0, CHUNK, zrow, 0)
    for k in range(ROWS_PER_TILE // CHUNK):
        pltpu.sync_copy(
            rows.at[0],
            acc_sh.at[pl.ds(s * ROWS_PER_TILE + k * CHUNK, CHUNK)])
    plsc.subcore_barrier()

    def idx_wait(b):
        pltpu.make_async_copy(pairs_hbm.at[0], ibuf.at[b], isems[b]).wait()

    def gat_wait(b):
        pltpu.make_async_copy(g_hbm.at[pl.ds(0, CHUNK)], rows.at[b],
                              gsems[b]).wait()

    for b in range(NBUF):
        pltpu.async_copy(pairs_hbm.at[base + b], ibuf.at[b], isems[b])
    idx_wait(0)
    pltpu.async_copy(g_hbm.at[ibuf.at[0, 0]], rows.at[0], gsems[0])

    def body(jj, carry):
        for b in range(NBUF):
            j = jj * NBUF + b
            b1 = (b + 1) % NBUF
            gat_wait(b)

            @pl.when(j + 1 < my_n)
            def _():
                idx_wait(b1)
                pltpu.async_copy(g_hbm.at[ibuf.at[b1, 0]], rows.at[b1],
                                 gsems[b1])

            pltpu.sync_copy(rows.at[b], acc_sh.at[ibuf.at[b, 1]], add=True)

            @pl.when(j + NBUF < my_n)
            def _():
                pltpu.async_copy(pairs_hbm.at[base + j + NBUF], ibuf.at[b],
                                 isems[b])
        return carry

    lax.fori_loop(0, my_n // NBUF, body, 0)
    plsc.subcore_barrier()
    for k in range(ROWS_PER_TILE // CHUNK):
        b = k % NBUF
        r0 = s * ROWS_PER_TILE + k * CHUNK
        pltpu.sync_copy(acc_sh.at[pl.ds(r0, CHUNK)], rows.at[b])
        pltpu.sync_copy(rows.at[b], out_hbm.at[pl.ds(c * N_PAD + r0, CHUNK)])


def _deg_call(pairs, nchunk):
    k = pl.kernel(
        functools.partial(_deg_body, nchunk),
        out_type=jax.ShapeDtypeStruct((NW, N_PAD), jnp.float32),
        mesh=_mesh(),
        scratch_types=[
            pltpu.VMEM((nchunk, 2, CHUNK), jnp.int32),
            pltpu.VMEM((N_PAD,), jnp.float32),
        ],
        compiler_params=pltpu.CompilerParams(needs_layout_passes=False),
    )
    return k(pairs)


def _gather_call(pairs, g, n0, n1):
    k = pl.kernel(
        functools.partial(_gather_body, n0, n1),
        out_type=jax.ShapeDtypeStruct((NC * N_PAD, DIM), jnp.float32),
        mesh=_mesh(),
        scratch_types=[
            pltpu.VMEM((NBUF, 2, CHUNK), jnp.int32),
            pltpu.VMEM((NBUF, CHUNK, DIM), jnp.float32),
            pltpu.VMEM_SHARED((N_PAD, DIM), jnp.float32),
        ] + [pltpu.SemaphoreType.DMA] * (2 * NBUF),
    )
    return k(pairs, g)


def _dinv(cnt):
    deg = jnp.sum(cnt, axis=0) + 1.0
    return lax.rsqrt(deg)[:, None]


def _tc1_body(x_ref, w1_ref, cnt_ref, g_ref):
    dinv = _dinv(cnt_ref[...])
    g_ref[...] = jnp.dot(x_ref[...], w1_ref[...],
                         preferred_element_type=jnp.float32) * dinv


def _tc2_body(acc_ref, g1_ref, cnt_ref, w2_ref, b1_ref, g2_ref):
    dinv = _dinv(cnt_ref[...])
    a = acc_ref[0] + acc_ref[1] + g1_ref[...]
    h = jnp.maximum(a * dinv + b1_ref[...], 0.0)
    g2_ref[...] = jnp.dot(h, w2_ref[...],
                          preferred_element_type=jnp.float32) * dinv


def _tc3_body(acc_ref, g2_ref, cnt_ref, b2_ref, nb_ref, pv_ref, wn_ref, wt_ref,
              out_ref):
    dinv = _dinv(cnt_ref[...])
    a = acc_ref[0] + acc_ref[1] + g2_ref[...]
    out_ref[...] = (a * dinv + b2_ref[...]
                    + jnp.dot(nb_ref[...], wn_ref[...],
                              preferred_element_type=jnp.float32)
                    + jnp.dot(pv_ref[...], wt_ref[...],
                              preferred_element_type=jnp.float32))


_ROW = pl.BlockSpec((BLK, DIM), lambda i: (i, 0))
_MAT = pl.BlockSpec((DIM, DIM), lambda i: (0, 0))
_CNT = pl.BlockSpec((NW, BLK), lambda i: (0, i))
_ACC = pl.BlockSpec((2, BLK, DIM), lambda i: (0, i, 0))
_BIAS = pl.BlockSpec((1, DIM), lambda i: (0, 0))
_GRID = (N_PAD // BLK,)
_OUT_ROWS = jax.ShapeDtypeStruct((N_PAD, DIM), jnp.float32)


def _tc1(x_pad, W1, counts):
    return pl.pallas_call(
        _tc1_body, grid=_GRID,
        in_specs=[_ROW, _MAT, _CNT], out_specs=_ROW,
        out_shape=_OUT_ROWS)(x_pad, W1, counts)


def _tc2(acc1, g1, counts, W2, b1):
    return pl.pallas_call(
        _tc2_body, grid=_GRID,
        in_specs=[_ACC, _ROW, _CNT, _MAT, _BIAS], out_specs=_ROW,
        out_shape=_OUT_ROWS)(acc1, g1, counts, W2, b1)


def _tc3(acc2, g2, counts, b2, nb, pv, Wn, Wt):
    return pl.pallas_call(
        _tc3_body, grid=_GRID,
        in_specs=[_ACC, _ROW, _CNT, _BIAS, _ROW, _ROW, _MAT, _MAT],
        out_specs=_ROW,
        out_shape=_OUT_ROWS)(acc2, g2, counts, b2, nb, pv, Wn, Wt)


def kernel(x, edge_index, neighbor_features, prev_time_features,
           W1, b1, W2, b2, Wn, Wt):
    E = edge_index.shape[1]
    tch = -(-(-(-E // CHUNK)) // (4 * NW)) * 4 * NW
    nchunk = tch // NW
    n1 = 2 * (-(-(3 * tch) // (20 * NS)) // 2)
    n0 = tch // NS - n1
    e_pad = tch * CHUNK
    ei = edge_index.astype(jnp.int32)
    pad = jnp.full((e_pad - E,), N_NODES, jnp.int32)
    src2 = jnp.concatenate([ei[0], pad]).reshape(tch, CHUNK)
    dst2 = jnp.concatenate([ei[1], pad]).reshape(tch, CHUNK)
    pairs = jnp.stack([src2, dst2], axis=1)

    rpad = ((0, N_PAD - N_NODES), (0, 0))
    x_pad = jnp.pad(x, rpad)
    nb_pad = jnp.pad(neighbor_features, rpad)
    pv_pad = jnp.pad(prev_time_features, rpad)
    counts = _deg_call(pairs, nchunk)
    g1 = _tc1(x_pad, W1, counts)
    acc1 = _gather_call(pairs, g1, n0, n1).reshape(NC, N_PAD, DIM)
    g2 = _tc2(acc1, g1, counts, W2, b1.reshape(1, DIM))
    acc2 = _gather_call(pairs, g2, n0, n1).reshape(NC, N_PAD, DIM)
    out = _tc3(acc2, g2, counts, b2.reshape(1, DIM), nb_pad, pv_pad, Wn, Wt)
    return out[:N_NODES]
